# baseline (speedup 1.0000x reference)
import jax
import jax.numpy as jnp
from jax import lax
from jax.experimental import pallas as pl
from jax.experimental.pallas import tpu as pltpu
from jax.experimental.pallas import tpu_sc as plsc

B = 16384
D = 32
NG = 20
NO = 21

NC = 2
NS = 16
NW = NC * NS
BW = B // NW
NCHUNK = 4
CH = BW // NCHUNK
GROUPS = BW // 16


def _sc_body(uid_hbm, iid_hbm, occ_hbm, genre_hbm,
             uemb_hbm, ubias_hbm, iemb_hbm, ibias_hbm,
             oemb_hbm, obias_hbm, gemb_hbm, gbias_hbm,
             out_hbm,
             uidx_v, iidx_v, occ_v, genre_v,
             urows_v, irows_v, ubias_v, ibias_v,
             oemb_v, obias_v, gemb_v, gbias_v, out_v,
             sem):
  wid = lax.axis_index("s") * NC + lax.axis_index("c")

  pltpu.sync_copy(uid_hbm.at[wid], uidx_v)
  pltpu.sync_copy(iid_hbm.at[wid], iidx_v)
  pltpu.sync_copy(occ_hbm.at[wid], occ_v)
  pltpu.sync_copy(genre_hbm.at[wid], genre_v)
  pltpu.sync_copy(oemb_hbm, oemb_v)
  pltpu.sync_copy(obias_hbm, obias_v)
  pltpu.sync_copy(gemb_hbm, gemb_v)
  pltpu.sync_copy(gbias_hbm, gbias_v)

  copies = []
  for j in range(NCHUNK):
    sl = pl.ds(j * CH, CH)
    copies.append(pltpu.async_copy(uemb_hbm.at[uidx_v.at[j]], urows_v.at[sl], sem))
    copies.append(pltpu.async_copy(iemb_hbm.at[iidx_v.at[j]], irows_v.at[sl], sem))
    copies.append(pltpu.async_copy(ubias_hbm.at[uidx_v.at[j]], ubias_v.at[sl], sem))
    copies.append(pltpu.async_copy(ibias_hbm.at[iidx_v.at[j]], ibias_v.at[sl], sem))

  iota = lax.iota(jnp.int32, 16)

  glo = iota
  ghi = jnp.minimum(iota + 16, NG - 1)
  rs_lo = jnp.zeros((16,), jnp.float32)
  rs_hi = jnp.zeros((16,), jnp.float32)
  for d in range(D):
    dfull = jnp.full((16,), d, jnp.int32)
    rs_lo = rs_lo + plsc.load_gather(gemb_v, [glo, dfull])
    rs_hi = rs_hi + plsc.load_gather(gemb_v, [ghi, dfull])
  rb_lo = gbias_v[0:16]
  rb_hi = gbias_v[16:32]

  zero = jnp.zeros((16,), jnp.float32)
  rs_s = []
  rb_s = []
  for g in range(NG):
    src_rs = rs_lo if g < 16 else rs_hi
    src_rb = rb_lo if g < 16 else rb_hi
    lane = g if g < 16 else g - 16
    sel = iota == lane
    rs_s.append(jnp.sum(jnp.where(sel, src_rs, zero)))
    rb_s.append(jnp.sum(jnp.where(sel, src_rb, zero)))

  for c in copies:
    c.wait()

  inv_d = jnp.float32(1.0 / D)

  def group(i, carry):
    bidx = i * 16 + iota
    occ_ids = plsc.load_gather(occ_v, [bidx])
    acc = jnp.zeros((16,), jnp.float32)
    isum = jnp.zeros((16,), jnp.float32)
    for d in range(D):
      dfull = jnp.full((16,), d, jnp.int32)
      qv = plsc.load_gather(urows_v, [bidx, dfull])
      iv = plsc.load_gather(irows_v, [bidx, dfull])
      ov = plsc.load_gather(oemb_v, [occ_ids, dfull])
      acc = acc + qv * (iv + ov)
      isum = isum + iv
    cnt = jnp.zeros((16,), jnp.float32)
    tot = jnp.zeros((16,), jnp.float32)
    totb = jnp.zeros((16,), jnp.float32)
    for g in range(NG):
      gfull = jnp.full((16,), g, jnp.int32)
      m = plsc.load_gather(genre_v, [bidx, gfull]).astype(jnp.float32)
      cnt = cnt + m
      tot = tot + m * rs_s[g]
      totb = totb + m * rb_s[g]
    bq = plsc.load_gather(ubias_v, [bidx])
    bi = plsc.load_gather(ibias_v, [bidx])
    bo = plsc.load_gather(obias_v, [occ_ids])
    inv = 1.0 / cnt
    res = acc + (tot * inv) * inv_d * isum + bq + bi + bo + totb * inv
    plsc.store_scatter(out_v, [bidx], res)
    return carry

  lax.fori_loop(0, GROUPS, group, 0)
  pltpu.sync_copy(out_v, out_hbm.at[pl.ds(wid * BW, BW)])


def kernel(user_id, item_id, occupation, genre,
           user_emb, user_bias, item_emb, item_bias,
           occ_emb, occ_bias, genre_emb, genre_bias_emb):
  mesh = plsc.VectorSubcoreMesh(core_axis_name="c", subcore_axis_name="s")
  f32 = jnp.float32
  i32 = jnp.int32
  k = pl.kernel(
      _sc_body,
      mesh=mesh,
      compiler_params=pltpu.CompilerParams(
          needs_layout_passes=False, use_tc_tiling_on_sc=False),
      out_type=jax.ShapeDtypeStruct((B,), f32),
      scratch_types=[
          pltpu.VMEM((NCHUNK, CH), i32),
          pltpu.VMEM((NCHUNK, CH), i32),
          pltpu.VMEM((BW,), i32),
          pltpu.VMEM((BW, NG), i32),
          pltpu.VMEM((BW, D), f32),
          pltpu.VMEM((BW, D), f32),
          pltpu.VMEM((BW,), f32),
          pltpu.VMEM((BW,), f32),
          pltpu.VMEM((NO, D), f32),
          pltpu.VMEM((NO,), f32),
          pltpu.VMEM((NG, D), f32),
          pltpu.VMEM((32,), f32),
          pltpu.VMEM((BW,), f32),
          pltpu.SemaphoreType.DMA,
      ],
  )
  gbias_pad = jnp.pad(genre_bias_emb.reshape(-1), (0, 32 - NG))
  return k(user_id.reshape(NW, NCHUNK, CH),
           item_id.reshape(NW, NCHUNK, CH),
           occupation.reshape(NW, BW),
           genre.reshape(NW, BW, NG),
           user_emb, user_bias.reshape(-1),
           item_emb, item_bias.reshape(-1),
           occ_emb, occ_bias.reshape(-1),
           genre_emb, gbias_pad)
